# Initial kernel scaffold; baseline (speedup 1.0000x reference)
import jax
import jax.numpy as jnp
from jax.experimental import pallas as pl


def kernel(x, W):
    raise NotImplementedError("write your pallas kernel here")


# speedup vs baseline: 1.8694x; 1.8694x over previous
import functools

import jax
import jax.numpy as jnp
from jax import lax
from jax.experimental import pallas as pl
from jax.experimental.pallas import tpu as pltpu
from jax.experimental.pallas import tpu_sc as plsc

DIM = 768
NC, NS = 2, 16
NW = NC * NS
K = 128


@functools.partial(jax.jit, static_argnums=(2,))
def _sc_gather(W, idx, n_total):
    n_per_w = n_total // NW
    n_chunks = n_per_w // K
    mesh = plsc.VectorSubcoreMesh(core_axis_name="c", subcore_axis_name="s")

    @functools.partial(
        pl.kernel,
        mesh=mesh,
        out_type=jax.ShapeDtypeStruct((n_total, DIM), jnp.float32),
        scratch_types=[
            pltpu.VMEM((n_per_w,), jnp.int32),
            pltpu.VMEM((K, DIM), jnp.float32),
            pltpu.SemaphoreType.DMA,
        ],
    )
    def k(W_hbm, idx_hbm, out_hbm, idx_v, rows_v, sem):
        wid = lax.axis_index("s") * NC + lax.axis_index("c")
        base = wid * n_per_w
        pltpu.sync_copy(idx_hbm.at[pl.ds(base, n_per_w)], idx_v)

        @pl.loop(0, n_chunks)
        def _chunk(j):
            pltpu.async_copy(
                W_hbm.at[idx_v.at[pl.ds(j * K, K)]], rows_v, sem).wait()
            pltpu.sync_copy(rows_v, out_hbm.at[pl.ds(base + j * K, K)])

    return k(W, idx)


def kernel(x, W):
    B, S = x.shape
    n_total = B * S
    out = _sc_gather(W, x.reshape(n_total), n_total)
    return out.reshape(B, S, DIM)


# speedup vs baseline: 1.9167x; 1.0253x over previous
import functools

import jax
import jax.numpy as jnp
from jax import lax
from jax.experimental import pallas as pl
from jax.experimental.pallas import tpu as pltpu
from jax.experimental.pallas import tpu_sc as plsc

DIM = 768
NC, NS = 2, 16
NW = NC * NS
K = 64
NBUF = 2


@functools.partial(jax.jit, static_argnums=(2,))
def _sc_gather(W, idx, n_total):
    n_per_w = n_total // NW
    n_chunks = n_per_w // K
    mesh = plsc.VectorSubcoreMesh(core_axis_name="c", subcore_axis_name="s")

    @functools.partial(
        pl.kernel,
        mesh=mesh,
        out_type=jax.ShapeDtypeStruct((n_total, DIM), jnp.float32),
        scratch_types=[
            pltpu.VMEM((n_per_w,), jnp.int32),
            pltpu.VMEM((NBUF, K, DIM), jnp.float32),
            [pltpu.SemaphoreType.DMA] * NBUF,
        ],
    )
    def k(W_hbm, idx_hbm, out_hbm, idx_v, rows_v, sems):
        wid = lax.axis_index("s") * NC + lax.axis_index("c")
        base = wid * n_per_w
        pltpu.sync_copy(idx_hbm.at[pl.ds(base, n_per_w)], idx_v)

        def gather(c, b):
            pltpu.async_copy(
                W_hbm.at[idx_v.at[pl.ds(c * K, K)]], rows_v.at[b], sems[b])

        for b in range(NBUF):
            gather(b, b)

        @pl.loop(0, n_chunks, step=NBUF)
        def _chunk(j):
            for b in range(NBUF):
                c = j + b
                pltpu.make_async_copy(
                    W_hbm.at[idx_v.at[pl.ds(c * K, K)]], rows_v.at[b],
                    sems[b]).wait()
                pltpu.sync_copy(rows_v.at[b], out_hbm.at[pl.ds(base + c * K, K)])

                @pl.when(c + NBUF < n_chunks)
                def _():
                    gather(c + NBUF, b)

    return k(W, idx)


def kernel(x, W):
    B, S = x.shape
    n_total = B * S
    out = _sc_gather(W, x.reshape(n_total), n_total)
    return out.reshape(B, S, DIM)
